# baseline (speedup 1.0000x reference)
import jax
import jax.numpy as jnp
from jax.experimental import pallas as pl
from jax.experimental.pallas import tpu as pltpu

TILE_V = 2560


def _fused_body(t_ref, wp_ref, bp_ref, wfa_ref, wfb_ref, bf_ref, out_ref, x_ref):
    @pl.when(pl.program_id(0) == 0)
    def _proj():
        proj = jax.lax.dot_general(
            t_ref[...], wp_ref[...],
            (((1,), (1,)), ((), ())),
            preferred_element_type=jnp.float32)
        proj = proj + bp_ref[...]
        g = 0.5 * proj * (1.0 + jax.lax.erf(proj * 0.7071067811865476))
        x_ref[...] = g.astype(jnp.bfloat16)

    x = x_ref[...]
    wfa = wfa_ref[...].astype(jnp.bfloat16)
    acca = jax.lax.dot_general(
        x, wfa, (((1,), (1,)), ((), ())),
        preferred_element_type=jnp.float32)
    out_ref[:, :TILE_V] = acca + bf_ref[:, :TILE_V]
    wfb = wfb_ref[...].astype(jnp.bfloat16)
    accb = jax.lax.dot_general(
        x, wfb, (((1,), (1,)), ((), ())),
        preferred_element_type=jnp.float32)
    out_ref[:, TILE_V:] = accb + bf_ref[:, TILE_V:]


def kernel(t, Wp, bp, Wf, bf):
    B, S, H = t.shape
    P, _ = Wp.shape
    V, _ = Wf.shape
    M = B * S

    out = pl.pallas_call(
        _fused_body,
        grid=(pl.cdiv(V, 2 * TILE_V),),
        in_specs=[
            pl.BlockSpec((M, H), lambda v: (0, 0)),
            pl.BlockSpec((P, H), lambda v: (0, 0)),
            pl.BlockSpec((1, P), lambda v: (0, 0)),
            pl.BlockSpec((TILE_V, P), lambda v: (2 * v, 0)),
            pl.BlockSpec((TILE_V, P), lambda v: (2 * v + 1, 0)),
            pl.BlockSpec((1, 2 * TILE_V), lambda v: (0, v)),
        ],
        out_specs=pl.BlockSpec((M, 2 * TILE_V), lambda v: (0, v)),
        out_shape=jax.ShapeDtypeStruct((M, V), jnp.float32),
        scratch_shapes=[pltpu.VMEM((M, P), jnp.bfloat16)],
        compiler_params=pltpu.CompilerParams(
            dimension_semantics=("arbitrary",)),
    )(t.reshape(M, H), Wp, bp.reshape(1, P), Wf, Wf, bf.reshape(1, V))
    return out.reshape(B, S, V)


# speedup vs baseline: 1.0012x; 1.0012x over previous
import jax
import jax.numpy as jnp
from jax.experimental import pallas as pl
from jax.experimental.pallas import tpu as pltpu

TILE_V = 5120


def _fused_body(t_ref, wp_ref, bp_ref, wf_ref, bf_ref, out_ref, x_ref):
    @pl.when(pl.program_id(0) == 0)
    def _proj():
        proj = jax.lax.dot_general(
            t_ref[...], wp_ref[...],
            (((1,), (1,)), ((), ())),
            preferred_element_type=jnp.float32)
        proj = proj + bp_ref[...]
        g = 0.5 * proj * (1.0 + jax.lax.erf(proj * 0.7071067811865476))
        x_ref[...] = g.astype(jnp.bfloat16)

    wf = wf_ref[...].astype(jnp.bfloat16)
    acc = jax.lax.dot_general(
        x_ref[...], wf,
        (((1,), (1,)), ((), ())),
        preferred_element_type=jnp.float32)
    out_ref[...] = acc + bf_ref[...]


def kernel(t, Wp, bp, Wf, bf):
    B, S, H = t.shape
    P, _ = Wp.shape
    V, _ = Wf.shape
    M = B * S

    out = pl.pallas_call(
        _fused_body,
        grid=(pl.cdiv(V, TILE_V),),
        in_specs=[
            pl.BlockSpec((M, H), lambda v: (0, 0)),
            pl.BlockSpec((P, H), lambda v: (0, 0)),
            pl.BlockSpec((1, P), lambda v: (0, 0)),
            pl.BlockSpec((TILE_V, P), lambda v: (v, 0)),
            pl.BlockSpec((1, TILE_V), lambda v: (0, v)),
        ],
        out_specs=pl.BlockSpec((M, TILE_V), lambda v: (0, v)),
        out_shape=jax.ShapeDtypeStruct((M, V), jnp.float32),
        scratch_shapes=[pltpu.VMEM((M, P), jnp.bfloat16)],
        compiler_params=pltpu.CompilerParams(
            dimension_semantics=("arbitrary",)),
    )(t.reshape(M, H), Wp, bp.reshape(1, P), Wf, bf.reshape(1, V))
    return out.reshape(B, S, V)
